# baseline (speedup 1.0000x reference)
import functools

import jax
import jax.numpy as jnp
from jax.experimental import pallas as pl
from jax.experimental.pallas import tpu as pltpu

NUM_GATES = 16
DIM = 4096
EPS = 1e-9
SECOND_THRESHOLD = 0.2
CAPACITY = 160
BN = 256
BR = BN * NUM_GATES


def _gating_kernel(x_ref, w_ref, p_ref, disp_hbm, comb_hbm, bal_ref, z_ref,
                   carry_ref, proxy_ref, accb_ref, accz_ref, rep_ref, tri_ref,
                   comb_buf, disp_buf, sem, *, nb_total):
    b = pl.program_id(0)
    nb = pl.program_id(1)
    step = b * nb_total + nb
    total = 4 * nb_total
    slot = step % 2

    @pl.when(step == 0)
    def _reset_all():
        accb_ref[...] = jnp.zeros_like(accb_ref)
        accz_ref[...] = jnp.zeros_like(accz_ref)
        rep_ref[...] = (
            (jax.lax.broadcasted_iota(jnp.int32, (BR, BN), 0) // NUM_GATES)
            == jax.lax.broadcasted_iota(jnp.int32, (BR, BN), 1)
        ).astype(jnp.bfloat16)
        tri_ref[...] = (
            jax.lax.broadcasted_iota(jnp.int32, (BN, BN), 0)
            > jax.lax.broadcasted_iota(jnp.int32, (BN, BN), 1)
        ).astype(jnp.bfloat16)

    @pl.when(nb == 0)
    def _reset_batch():
        carry_ref[...] = jnp.zeros_like(carry_ref)
        proxy_ref[...] = jnp.zeros_like(proxy_ref)

    @pl.when(step >= 2)
    def _drain_prev():
        pltpu.make_async_copy(
            comb_buf.at[slot], comb_hbm.at[b, pl.ds(nb * BN, BN)],
            sem.at[slot, 0]).wait()
        pltpu.make_async_copy(
            disp_buf.at[slot], disp_hbm.at[b, pl.ds(nb * BN, BN)],
            sem.at[slot, 1]).wait()

    xb = x_ref[0]
    logits = jax.lax.dot_general(
        xb, w_ref[...], (((1,), (0,)), ((), ())),
        preferred_element_type=jnp.float32)

    m = jnp.max(logits, axis=1, keepdims=True)
    ex = jnp.exp(logits - m)
    s = jnp.sum(ex, axis=1, keepdims=True)
    sm = ex / s
    lse = m + jnp.log(s)

    accz_ref[...] = accz_ref[...] + jnp.sum(lse, axis=(0, 1), keepdims=True)
    proxy_ref[...] = proxy_ref[...] + jnp.sum(sm, axis=0, keepdims=True)

    e_iota = jax.lax.broadcasted_iota(jnp.int32, (BN, NUM_GATES), 1)
    g1 = jnp.max(sm, axis=1, keepdims=True)
    i1 = jnp.min(jnp.where(sm == g1, e_iota, NUM_GATES), axis=1, keepdims=True)
    sm2 = jnp.where(e_iota == i1, -jnp.inf, sm)
    g2 = jnp.max(sm2, axis=1, keepdims=True)
    i2 = jnp.min(jnp.where(sm2 == g2, e_iota, NUM_GATES), axis=1, keepdims=True)

    denom = g1 + g2 + EPS
    g1n = g1 / denom
    g2n = g2 / denom

    probs = p_ref[0, 0]
    keep2 = probs < (g2n / jnp.float32(SECOND_THRESHOLD))

    mask1 = (e_iota == i1).astype(jnp.bfloat16)
    mask2 = ((e_iota == i2) & keep2).astype(jnp.bfloat16)

    excl1 = jax.lax.dot_general(
        tri_ref[...], mask1, (((1,), (0,)), ((), ())),
        preferred_element_type=jnp.float32)
    excl2 = jax.lax.dot_general(
        tri_ref[...], mask2, (((1,), (0,)), ((), ())),
        preferred_element_type=jnp.float32)

    mask1f = mask1.astype(jnp.float32)
    mask2f = mask2.astype(jnp.float32)
    carry1 = carry_ref[0:1, :]
    carry2 = carry_ref[1:2, :]
    pos1 = jnp.sum((excl1 + carry1) * mask1f, axis=1, keepdims=True)
    pos2 = jnp.sum((excl2 + carry2) * mask2f, axis=1, keepdims=True)
    carry_ref[0:1, :] = carry1 + jnp.sum(mask1f, axis=0, keepdims=True)
    carry_ref[1:2, :] = carry2 + jnp.sum(mask2f, axis=0, keepdims=True)

    kept1 = (pos1 < CAPACITY).astype(jnp.float32)
    kept2 = (keep2 & (pos2 < CAPACITY)).astype(jnp.float32)
    g1f = g1n * kept1
    g2f = g2n * kept2

    p1c = jnp.minimum(pos1, jnp.float32(CAPACITY))
    p2c = jnp.minimum(pos2, jnp.float32(CAPACITY))
    vals = jnp.concatenate(
        [i1.astype(jnp.float32), p1c, g1f,
         i2.astype(jnp.float32), p2c, g2f], axis=1)
    rv = jax.lax.dot_general(
        rep_ref[...], vals.astype(jnp.bfloat16), (((1,), (0,)), ((), ())),
        preferred_element_type=jnp.float32)
    i1r = jnp.round(rv[:, 0:1])
    p1r = jnp.round(rv[:, 1:2])
    g1r = rv[:, 2:3]
    i2r = jnp.round(rv[:, 3:4])
    p2r = jnp.round(rv[:, 4:5])
    g2r = rv[:, 5:6]

    e_row = (jax.lax.broadcasted_iota(jnp.int32, (BR, 1), 0)
             & (NUM_GATES - 1)).astype(jnp.float32)
    val_rows = (jnp.where(e_row == i1r, g1r, 0.0)
                + jnp.where(e_row == i2r, g2r, 0.0))
    pos_rows = jnp.where(e_row == i1r, p1r, jnp.float32(4096.0))
    pos_rows = jnp.where(e_row == i2r, p2r, pos_rows)

    c_iota = jax.lax.broadcasted_iota(jnp.int32, (BR, CAPACITY), 1)
    combine = jnp.where(c_iota.astype(jnp.float32) == pos_rows, val_rows, 0.0)
    comb_buf[slot] = combine.reshape(BN, NUM_GATES, CAPACITY)
    disp_buf[slot] = (combine != 0.0).astype(jnp.float32).reshape(
        BN, NUM_GATES, CAPACITY)

    cp_c = pltpu.make_async_copy(
        comb_buf.at[slot], comb_hbm.at[b, pl.ds(nb * BN, BN)], sem.at[slot, 0])
    cp_d = pltpu.make_async_copy(
        disp_buf.at[slot], disp_hbm.at[b, pl.ds(nb * BN, BN)], sem.at[slot, 1])
    cp_c.start()
    cp_d.start()

    @pl.when(step == total - 1)
    def _drain_tail():
        other = (slot + 1) % 2
        pltpu.make_async_copy(
            comb_buf.at[other], comb_hbm.at[b, pl.ds(nb * BN, BN)],
            sem.at[other, 0]).wait()
        pltpu.make_async_copy(
            disp_buf.at[other], disp_hbm.at[b, pl.ds(nb * BN, BN)],
            sem.at[other, 1]).wait()
        cp_c.wait()
        cp_d.wait()

    @pl.when(nb == nb_total - 1)
    def _finish_batch():
        accb_ref[...] = accb_ref[...] + jnp.sum(
            proxy_ref[...] * carry_ref[0:1, :], axis=(0, 1), keepdims=True)

    bal_ref[...] = accb_ref[...] * jnp.float32(4.0 / (2048.0 * 2048.0))
    z_ref[...] = accz_ref[...] * jnp.float32(0.25)


@jax.jit
def kernel(x, w_gating):
    b, n, d = x.shape
    nb_total = n // BN
    probs = jax.lax.stop_gradient(
        jax.random.uniform(jax.random.key(42), (b, n), dtype=jnp.float32))
    probs4 = probs.reshape(b, nb_total, BN, 1)

    grid = (b, nb_total)
    out_shape = [
        jax.ShapeDtypeStruct((b, n, NUM_GATES, CAPACITY), jnp.float32),
        jax.ShapeDtypeStruct((b, n, NUM_GATES, CAPACITY), jnp.float32),
        jax.ShapeDtypeStruct((1, 1), jnp.float32),
        jax.ShapeDtypeStruct((1, 1), jnp.float32),
    ]
    disp, comb, bal, z = pl.pallas_call(
        functools.partial(_gating_kernel, nb_total=nb_total),
        grid=grid,
        in_specs=[
            pl.BlockSpec((1, BN, d), lambda i, j: (i, j, 0)),
            pl.BlockSpec((d, NUM_GATES), lambda i, j: (0, 0)),
            pl.BlockSpec((1, 1, BN, 1), lambda i, j: (i, j, 0, 0)),
        ],
        out_specs=[
            pl.BlockSpec(memory_space=pl.ANY),
            pl.BlockSpec(memory_space=pl.ANY),
            pl.BlockSpec((1, 1), lambda i, j: (0, 0)),
            pl.BlockSpec((1, 1), lambda i, j: (0, 0)),
        ],
        out_shape=out_shape,
        scratch_shapes=[
            pltpu.VMEM((2, NUM_GATES), jnp.float32),
            pltpu.VMEM((1, NUM_GATES), jnp.float32),
            pltpu.VMEM((1, 1), jnp.float32),
            pltpu.VMEM((1, 1), jnp.float32),
            pltpu.VMEM((BR, BN), jnp.bfloat16),
            pltpu.VMEM((BN, BN), jnp.bfloat16),
            pltpu.VMEM((2, BN, NUM_GATES, CAPACITY), jnp.float32),
            pltpu.VMEM((2, BN, NUM_GATES, CAPACITY), jnp.float32),
            pltpu.SemaphoreType.DMA((2, 2)),
        ],
    )(x, w_gating, probs4)

    return disp, comb, bal[0, 0], z[0, 0]


# speedup vs baseline: 1.1397x; 1.1397x over previous
import functools

import jax
import jax.numpy as jnp
from jax.experimental import pallas as pl
from jax.experimental.pallas import tpu as pltpu

NUM_GATES = 16
DIM = 4096
EPS = 1e-9
SECOND_THRESHOLD = 0.2
CAPACITY = 160
BN = 256
BR = BN * NUM_GATES


def _gating_kernel(x_ref, w_ref, p_ref, disp_ref, comb_ref, bal_ref, z_ref,
                   carry_ref, proxy_ref, accb_ref, accz_ref, rep_ref, tri_ref,
                   *, nb_total):
    b = pl.program_id(0)
    nb = pl.program_id(1)
    step = b * nb_total + nb

    @pl.when(step == 0)
    def _reset_all():
        accb_ref[...] = jnp.zeros_like(accb_ref)
        accz_ref[...] = jnp.zeros_like(accz_ref)
        rep_ref[...] = (
            (jax.lax.broadcasted_iota(jnp.int32, (BR, BN), 0) // NUM_GATES)
            == jax.lax.broadcasted_iota(jnp.int32, (BR, BN), 1)
        ).astype(jnp.bfloat16)
        tri_ref[...] = (
            jax.lax.broadcasted_iota(jnp.int32, (BN, BN), 0)
            > jax.lax.broadcasted_iota(jnp.int32, (BN, BN), 1)
        ).astype(jnp.bfloat16)

    @pl.when(nb == 0)
    def _reset_batch():
        carry_ref[...] = jnp.zeros_like(carry_ref)
        proxy_ref[...] = jnp.zeros_like(proxy_ref)

    xb = x_ref[0]
    logits = jax.lax.dot_general(
        xb, w_ref[...], (((1,), (0,)), ((), ())),
        preferred_element_type=jnp.float32)

    m = jnp.max(logits, axis=1, keepdims=True)
    ex = jnp.exp(logits - m)
    s = jnp.sum(ex, axis=1, keepdims=True)
    sm = ex / s
    lse = m + jnp.log(s)

    accz_ref[...] = accz_ref[...] + jnp.sum(lse, axis=(0, 1), keepdims=True)
    proxy_ref[...] = proxy_ref[...] + jnp.sum(sm, axis=0, keepdims=True)

    e_iota = jax.lax.broadcasted_iota(jnp.int32, (BN, NUM_GATES), 1)
    g1 = jnp.max(sm, axis=1, keepdims=True)
    i1 = jnp.min(jnp.where(sm == g1, e_iota, NUM_GATES), axis=1, keepdims=True)
    sm2 = jnp.where(e_iota == i1, -jnp.inf, sm)
    g2 = jnp.max(sm2, axis=1, keepdims=True)
    i2 = jnp.min(jnp.where(sm2 == g2, e_iota, NUM_GATES), axis=1, keepdims=True)

    denom = g1 + g2 + EPS
    g1n = g1 / denom
    g2n = g2 / denom

    probs = p_ref[0, 0]
    keep2 = probs < (g2n / jnp.float32(SECOND_THRESHOLD))

    mask1 = (e_iota == i1).astype(jnp.bfloat16)
    mask2 = ((e_iota == i2) & keep2).astype(jnp.bfloat16)

    excl1 = jax.lax.dot_general(
        tri_ref[...], mask1, (((1,), (0,)), ((), ())),
        preferred_element_type=jnp.float32)
    excl2 = jax.lax.dot_general(
        tri_ref[...], mask2, (((1,), (0,)), ((), ())),
        preferred_element_type=jnp.float32)

    mask1f = mask1.astype(jnp.float32)
    mask2f = mask2.astype(jnp.float32)
    carry1 = carry_ref[0:1, :]
    carry2 = carry_ref[1:2, :]
    pos1 = jnp.sum((excl1 + carry1) * mask1f, axis=1, keepdims=True)
    pos2 = jnp.sum((excl2 + carry2) * mask2f, axis=1, keepdims=True)
    carry_ref[0:1, :] = carry1 + jnp.sum(mask1f, axis=0, keepdims=True)
    carry_ref[1:2, :] = carry2 + jnp.sum(mask2f, axis=0, keepdims=True)

    kept1 = (pos1 < CAPACITY).astype(jnp.float32)
    kept2 = (keep2 & (pos2 < CAPACITY)).astype(jnp.float32)
    g1f = g1n * kept1
    g2f = g2n * kept2

    p1c = jnp.minimum(pos1, jnp.float32(CAPACITY))
    p2c = jnp.minimum(pos2, jnp.float32(CAPACITY))
    vals = jnp.concatenate(
        [i1.astype(jnp.float32), p1c, g1f,
         i2.astype(jnp.float32), p2c, g2f], axis=1)
    rv = jax.lax.dot_general(
        rep_ref[...], vals.astype(jnp.bfloat16), (((1,), (0,)), ((), ())),
        preferred_element_type=jnp.float32)
    i1r = jnp.round(rv[:, 0:1])
    p1r = jnp.round(rv[:, 1:2])
    g1r = rv[:, 2:3]
    i2r = jnp.round(rv[:, 3:4])
    p2r = jnp.round(rv[:, 4:5])
    g2r = rv[:, 5:6]

    e_row = (jax.lax.broadcasted_iota(jnp.int32, (BR, 1), 0)
             & (NUM_GATES - 1)).astype(jnp.float32)
    val_rows = (jnp.where(e_row == i1r, g1r, 0.0)
                + jnp.where(e_row == i2r, g2r, 0.0))
    pos_rows = jnp.where(e_row == i1r, p1r, jnp.float32(4096.0))
    pos_rows = jnp.where(e_row == i2r, p2r, pos_rows)

    c_iota = jax.lax.broadcasted_iota(jnp.int32, (BR, CAPACITY), 1)
    combine = jnp.where(c_iota.astype(jnp.float32) == pos_rows, val_rows, 0.0)
    comb_ref[...] = combine
    disp_ref[...] = (combine != 0.0).astype(jnp.float32)

    @pl.when(nb == nb_total - 1)
    def _finish_batch():
        accb_ref[...] = accb_ref[...] + jnp.sum(
            proxy_ref[...] * carry_ref[0:1, :], axis=(0, 1), keepdims=True)

    bal_ref[...] = accb_ref[...] * jnp.float32(4.0 / (2048.0 * 2048.0))
    z_ref[...] = accz_ref[...] * jnp.float32(0.25)


@jax.jit
def kernel(x, w_gating):
    b, n, d = x.shape
    nb_total = n // BN
    probs = jax.lax.stop_gradient(
        jax.random.uniform(jax.random.key(42), (b, n), dtype=jnp.float32))
    probs4 = probs.reshape(b, nb_total, BN, 1)

    grid = (b, nb_total)
    out_shape = [
        jax.ShapeDtypeStruct((b * n * NUM_GATES, CAPACITY), jnp.float32),
        jax.ShapeDtypeStruct((b * n * NUM_GATES, CAPACITY), jnp.float32),
        jax.ShapeDtypeStruct((1, 1), jnp.float32),
        jax.ShapeDtypeStruct((1, 1), jnp.float32),
    ]
    disp, comb, bal, z = pl.pallas_call(
        functools.partial(_gating_kernel, nb_total=nb_total),
        grid=grid,
        in_specs=[
            pl.BlockSpec((1, BN, d), lambda i, j: (i, j, 0)),
            pl.BlockSpec((d, NUM_GATES), lambda i, j: (0, 0)),
            pl.BlockSpec((1, 1, BN, 1), lambda i, j: (i, j, 0, 0)),
        ],
        out_specs=[
            pl.BlockSpec((BR, CAPACITY), lambda i, j, nbt=nb_total: (i * nbt + j, 0)),
            pl.BlockSpec((BR, CAPACITY), lambda i, j, nbt=nb_total: (i * nbt + j, 0)),
            pl.BlockSpec((1, 1), lambda i, j: (0, 0)),
            pl.BlockSpec((1, 1), lambda i, j: (0, 0)),
        ],
        out_shape=out_shape,
        scratch_shapes=[
            pltpu.VMEM((2, NUM_GATES), jnp.float32),
            pltpu.VMEM((1, NUM_GATES), jnp.float32),
            pltpu.VMEM((1, 1), jnp.float32),
            pltpu.VMEM((1, 1), jnp.float32),
            pltpu.VMEM((BR, BN), jnp.bfloat16),
            pltpu.VMEM((BN, BN), jnp.bfloat16),
        ],
    )(x, w_gating, probs4)

    dispatch = disp.reshape(b, n, NUM_GATES, CAPACITY)
    combine = comb.reshape(b, n, NUM_GATES, CAPACITY)
    return dispatch, combine, bal[0, 0], z[0, 0]


# speedup vs baseline: 1.7129x; 1.5029x over previous
import functools

import jax
import jax.numpy as jnp
from jax.experimental import pallas as pl
from jax.experimental.pallas import tpu as pltpu

NUM_GATES = 16
DIM = 4096
EPS = 1e-9
SECOND_THRESHOLD = 0.2
CAPACITY = 160
BN = 256
BR = BN * NUM_GATES


def _gating_kernel(x_ref, w_ref, p_ref, disp_ref, comb_ref, bal_ref, z_ref,
                   carry_ref, proxy_ref, accb_ref, accz_ref, tri_ref,
                   *, nb_total):
    b = pl.program_id(0)
    nb = pl.program_id(1)
    step = b * nb_total + nb

    @pl.when(step == 0)
    def _reset_all():
        accb_ref[...] = jnp.zeros_like(accb_ref)
        accz_ref[...] = jnp.zeros_like(accz_ref)
        tri_ref[...] = (
            jax.lax.broadcasted_iota(jnp.int32, (BN, BN), 0)
            > jax.lax.broadcasted_iota(jnp.int32, (BN, BN), 1)
        ).astype(jnp.bfloat16)

    @pl.when(nb == 0)
    def _reset_batch():
        carry_ref[...] = jnp.zeros_like(carry_ref)
        proxy_ref[...] = jnp.zeros_like(proxy_ref)

    xb = x_ref[0]
    logits = jax.lax.dot_general(
        xb, w_ref[...], (((1,), (0,)), ((), ())),
        preferred_element_type=jnp.float32)

    m = jnp.max(logits, axis=1, keepdims=True)
    ex = jnp.exp(logits - m)
    s = jnp.sum(ex, axis=1, keepdims=True)
    sm = ex / s
    lse = m + jnp.log(s)

    accz_ref[...] = accz_ref[...] + jnp.sum(lse, axis=(0, 1), keepdims=True)
    proxy_ref[...] = proxy_ref[...] + jnp.sum(sm, axis=0, keepdims=True)

    e_iota = jax.lax.broadcasted_iota(jnp.int32, (BN, NUM_GATES), 1)
    g1 = jnp.max(sm, axis=1, keepdims=True)
    i1 = jnp.min(jnp.where(sm == g1, e_iota, NUM_GATES), axis=1, keepdims=True)
    sm2 = jnp.where(e_iota == i1, -jnp.inf, sm)
    g2 = jnp.max(sm2, axis=1, keepdims=True)
    i2 = jnp.min(jnp.where(sm2 == g2, e_iota, NUM_GATES), axis=1, keepdims=True)

    denom = g1 + g2 + EPS
    g1n = g1 / denom
    g2n = g2 / denom

    probs = p_ref[0, 0]
    keep2 = probs < (g2n / jnp.float32(SECOND_THRESHOLD))

    mask1 = (e_iota == i1).astype(jnp.bfloat16)
    mask2 = ((e_iota == i2) & keep2).astype(jnp.bfloat16)

    excl1 = jax.lax.dot_general(
        tri_ref[...], mask1, (((1,), (0,)), ((), ())),
        preferred_element_type=jnp.float32)
    excl2 = jax.lax.dot_general(
        tri_ref[...], mask2, (((1,), (0,)), ((), ())),
        preferred_element_type=jnp.float32)

    mask1f = mask1.astype(jnp.float32)
    mask2f = mask2.astype(jnp.float32)
    carry1 = carry_ref[0:1, :]
    carry2 = carry_ref[1:2, :]
    pos1 = jnp.sum((excl1 + carry1) * mask1f, axis=1, keepdims=True)
    pos2 = jnp.sum((excl2 + carry2) * mask2f, axis=1, keepdims=True)
    carry_ref[0:1, :] = carry1 + jnp.sum(mask1f, axis=0, keepdims=True)
    carry_ref[1:2, :] = carry2 + jnp.sum(mask2f, axis=0, keepdims=True)

    kept1 = (pos1 < CAPACITY).astype(jnp.float32)
    kept2 = (keep2 & (pos2 < CAPACITY)).astype(jnp.float32)
    g1f = g1n * kept1
    g2f = g2n * kept2

    idx1 = i1 * CAPACITY + pos1.astype(jnp.int32)
    idx2 = i2 * CAPACITY + pos2.astype(jnp.int32)

    c_iota = jax.lax.broadcasted_iota(jnp.int32, (BN, NUM_GATES * CAPACITY), 1)
    combine = (jnp.where(c_iota == idx1, g1f, 0.0)
               + jnp.where(c_iota == idx2, g2f, 0.0))
    comb_ref[0] = combine
    disp_ref[0] = (combine != 0.0).astype(jnp.float32)

    @pl.when(nb == nb_total - 1)
    def _finish_batch():
        accb_ref[...] = accb_ref[...] + jnp.sum(
            proxy_ref[...] * carry_ref[0:1, :], axis=(0, 1), keepdims=True)

    bal_ref[...] = accb_ref[...] * jnp.float32(4.0 / (2048.0 * 2048.0))
    z_ref[...] = accz_ref[...] * jnp.float32(0.25)


@jax.jit
def kernel(x, w_gating):
    b, n, d = x.shape
    nb_total = n // BN
    probs = jax.lax.stop_gradient(
        jax.random.uniform(jax.random.key(42), (b, n), dtype=jnp.float32))
    probs4 = probs.reshape(b, nb_total, BN, 1)

    grid = (b, nb_total)
    flat = NUM_GATES * CAPACITY
    out_shape = [
        jax.ShapeDtypeStruct((b, n, flat), jnp.float32),
        jax.ShapeDtypeStruct((b, n, flat), jnp.float32),
        jax.ShapeDtypeStruct((1, 1), jnp.float32),
        jax.ShapeDtypeStruct((1, 1), jnp.float32),
    ]
    disp, comb, bal, z = pl.pallas_call(
        functools.partial(_gating_kernel, nb_total=nb_total),
        grid=grid,
        in_specs=[
            pl.BlockSpec((1, BN, d), lambda i, j: (i, j, 0)),
            pl.BlockSpec((d, NUM_GATES), lambda i, j: (0, 0)),
            pl.BlockSpec((1, 1, BN, 1), lambda i, j: (i, j, 0, 0)),
        ],
        out_specs=[
            pl.BlockSpec((1, BN, flat), lambda i, j: (i, j, 0)),
            pl.BlockSpec((1, BN, flat), lambda i, j: (i, j, 0)),
            pl.BlockSpec((1, 1), lambda i, j: (0, 0)),
            pl.BlockSpec((1, 1), lambda i, j: (0, 0)),
        ],
        out_shape=out_shape,
        scratch_shapes=[
            pltpu.VMEM((2, NUM_GATES), jnp.float32),
            pltpu.VMEM((1, NUM_GATES), jnp.float32),
            pltpu.VMEM((1, 1), jnp.float32),
            pltpu.VMEM((1, 1), jnp.float32),
            pltpu.VMEM((BN, BN), jnp.bfloat16),
        ],
    )(x, w_gating, probs4)

    dispatch = disp.reshape(b, n, NUM_GATES, CAPACITY)
    combine = comb.reshape(b, n, NUM_GATES, CAPACITY)
    return dispatch, combine, bal[0, 0], z[0, 0]


# speedup vs baseline: 1.8015x; 1.0517x over previous
import functools

import jax
import jax.numpy as jnp
from jax.experimental import pallas as pl
from jax.experimental.pallas import tpu as pltpu

NUM_GATES = 16
DIM = 4096
EPS = 1e-9
SECOND_THRESHOLD = 0.2
CAPACITY = 160
BN = 512
BR = BN * NUM_GATES


def _gating_kernel(x_ref, w_ref, p_ref, disp_ref, comb_ref, bal_ref, z_ref,
                   carry_ref, proxy_ref, accb_ref, accz_ref, tri_ref,
                   *, nb_total):
    b = pl.program_id(0)
    nb = pl.program_id(1)
    step = b * nb_total + nb

    @pl.when(step == 0)
    def _reset_all():
        accb_ref[...] = jnp.zeros_like(accb_ref)
        accz_ref[...] = jnp.zeros_like(accz_ref)
        tri_ref[...] = (
            jax.lax.broadcasted_iota(jnp.int32, (BN, BN), 0)
            > jax.lax.broadcasted_iota(jnp.int32, (BN, BN), 1)
        ).astype(jnp.bfloat16)

    @pl.when(nb == 0)
    def _reset_batch():
        carry_ref[...] = jnp.zeros_like(carry_ref)
        proxy_ref[...] = jnp.zeros_like(proxy_ref)

    xb = x_ref[0]
    logits = jax.lax.dot_general(
        xb, w_ref[...], (((1,), (0,)), ((), ())),
        preferred_element_type=jnp.float32)

    m = jnp.max(logits, axis=1, keepdims=True)
    ex = jnp.exp(logits - m)
    s = jnp.sum(ex, axis=1, keepdims=True)
    sm = ex / s
    lse = m + jnp.log(s)

    accz_ref[...] = accz_ref[...] + jnp.sum(lse, axis=(0, 1), keepdims=True)
    proxy_ref[...] = proxy_ref[...] + jnp.sum(sm, axis=0, keepdims=True)

    e_iota = jax.lax.broadcasted_iota(jnp.int32, (BN, NUM_GATES), 1)
    g1 = jnp.max(sm, axis=1, keepdims=True)
    i1 = jnp.min(jnp.where(sm == g1, e_iota, NUM_GATES), axis=1, keepdims=True)
    sm2 = jnp.where(e_iota == i1, -jnp.inf, sm)
    g2 = jnp.max(sm2, axis=1, keepdims=True)
    i2 = jnp.min(jnp.where(sm2 == g2, e_iota, NUM_GATES), axis=1, keepdims=True)

    denom = g1 + g2 + EPS
    g1n = g1 / denom
    g2n = g2 / denom

    probs = p_ref[0, 0]
    keep2 = probs < (g2n / jnp.float32(SECOND_THRESHOLD))

    mask1 = (e_iota == i1).astype(jnp.bfloat16)
    mask2 = ((e_iota == i2) & keep2).astype(jnp.bfloat16)

    excl1 = jax.lax.dot_general(
        tri_ref[...], mask1, (((1,), (0,)), ((), ())),
        preferred_element_type=jnp.float32)
    excl2 = jax.lax.dot_general(
        tri_ref[...], mask2, (((1,), (0,)), ((), ())),
        preferred_element_type=jnp.float32)

    mask1f = mask1.astype(jnp.float32)
    mask2f = mask2.astype(jnp.float32)
    carry1 = carry_ref[0:1, :]
    carry2 = carry_ref[1:2, :]
    pos1 = jnp.sum((excl1 + carry1) * mask1f, axis=1, keepdims=True)
    pos2 = jnp.sum((excl2 + carry2) * mask2f, axis=1, keepdims=True)
    carry_ref[0:1, :] = carry1 + jnp.sum(mask1f, axis=0, keepdims=True)
    carry_ref[1:2, :] = carry2 + jnp.sum(mask2f, axis=0, keepdims=True)

    kept1 = (pos1 < CAPACITY).astype(jnp.float32)
    kept2 = (keep2 & (pos2 < CAPACITY)).astype(jnp.float32)
    g1f = g1n * kept1
    g2f = g2n * kept2

    idx1 = i1 * CAPACITY + pos1.astype(jnp.int32)
    idx2 = i2 * CAPACITY + pos2.astype(jnp.int32)

    c_iota = jax.lax.broadcasted_iota(jnp.int32, (BN, NUM_GATES * CAPACITY), 1)
    combine = (jnp.where(c_iota == idx1, g1f, 0.0)
               + jnp.where(c_iota == idx2, g2f, 0.0))
    comb_ref[0] = combine
    disp_ref[0] = (combine != 0.0).astype(jnp.float32)

    @pl.when(nb == nb_total - 1)
    def _finish_batch():
        accb_ref[...] = accb_ref[...] + jnp.sum(
            proxy_ref[...] * carry_ref[0:1, :], axis=(0, 1), keepdims=True)

    bal_ref[...] = accb_ref[...] * jnp.float32(4.0 / (2048.0 * 2048.0))
    z_ref[...] = accz_ref[...] * jnp.float32(0.25)


@jax.jit
def kernel(x, w_gating):
    b, n, d = x.shape
    nb_total = n // BN
    probs = jax.lax.stop_gradient(
        jax.random.uniform(jax.random.key(42), (b, n), dtype=jnp.float32))
    probs4 = probs.reshape(b, nb_total, BN, 1)

    grid = (b, nb_total)
    flat = NUM_GATES * CAPACITY
    out_shape = [
        jax.ShapeDtypeStruct((b, n, flat), jnp.float32),
        jax.ShapeDtypeStruct((b, n, flat), jnp.float32),
        jax.ShapeDtypeStruct((1, 1), jnp.float32),
        jax.ShapeDtypeStruct((1, 1), jnp.float32),
    ]
    disp, comb, bal, z = pl.pallas_call(
        functools.partial(_gating_kernel, nb_total=nb_total),
        grid=grid,
        in_specs=[
            pl.BlockSpec((1, BN, d), lambda i, j: (i, j, 0)),
            pl.BlockSpec((d, NUM_GATES), lambda i, j: (0, 0)),
            pl.BlockSpec((1, 1, BN, 1), lambda i, j: (i, j, 0, 0)),
        ],
        out_specs=[
            pl.BlockSpec((1, BN, flat), lambda i, j: (i, j, 0)),
            pl.BlockSpec((1, BN, flat), lambda i, j: (i, j, 0)),
            pl.BlockSpec((1, 1), lambda i, j: (0, 0)),
            pl.BlockSpec((1, 1), lambda i, j: (0, 0)),
        ],
        out_shape=out_shape,
        scratch_shapes=[
            pltpu.VMEM((2, NUM_GATES), jnp.float32),
            pltpu.VMEM((1, NUM_GATES), jnp.float32),
            pltpu.VMEM((1, 1), jnp.float32),
            pltpu.VMEM((1, 1), jnp.float32),
            pltpu.VMEM((BN, BN), jnp.bfloat16),
        ],
    )(x, w_gating, probs4)

    dispatch = disp.reshape(b, n, NUM_GATES, CAPACITY)
    combine = comb.reshape(b, n, NUM_GATES, CAPACITY)
    return dispatch, combine, bal[0, 0], z[0, 0]
